# Initial kernel scaffold; baseline (speedup 1.0000x reference)
import jax
import jax.numpy as jnp
from jax.experimental import pallas as pl


def kernel(z):
    raise NotImplementedError("write your pallas kernel here")


# speedup vs baseline: 79.0211x; 79.0211x over previous
import jax
import jax.numpy as jnp
from jax.experimental import pallas as pl

_SOFT_DIM = 64
_TILE_N = 8192


def _triangle_kernel(z_ref, out_ref):
    z = z_ref[...]
    z_norm = (jnp.clip(z, -1.0, 1.0) + 1.0) * (0.5 * (_SOFT_DIM - 1))
    d = jax.lax.broadcasted_iota(jnp.int32, out_ref.shape, 1).astype(jnp.float32)
    out_ref[...] = jnp.maximum(1.0 - jnp.abs(z_norm - d), 0.0)


def kernel(z):
    B, _, N = z.shape
    tile = _TILE_N if N % _TILE_N == 0 else N
    return pl.pallas_call(
        _triangle_kernel,
        grid=(B, N // tile),
        in_specs=[pl.BlockSpec((1, 1, tile), lambda b, n: (b, 0, n))],
        out_specs=pl.BlockSpec((1, _SOFT_DIM, tile), lambda b, n: (b, 0, n)),
        out_shape=jax.ShapeDtypeStruct((B, _SOFT_DIM, N), z.dtype),
    )(z)


# speedup vs baseline: 112.9445x; 1.4293x over previous
import jax
import jax.numpy as jnp
from jax.experimental import pallas as pl

_SOFT_DIM = 64
_TILE_N = 32768


def _triangle_kernel(z_ref, out_ref):
    z = z_ref[...]
    z_norm = (jnp.clip(z, -1.0, 1.0) + 1.0) * (0.5 * (_SOFT_DIM - 1))
    d = jax.lax.broadcasted_iota(jnp.int32, out_ref.shape, 1).astype(jnp.float32)
    out_ref[...] = jnp.maximum(1.0 - jnp.abs(z_norm - d), 0.0)


def kernel(z):
    B, _, N = z.shape
    tile = _TILE_N if N % _TILE_N == 0 else N
    return pl.pallas_call(
        _triangle_kernel,
        grid=(B, N // tile),
        in_specs=[pl.BlockSpec((1, 1, tile), lambda b, n: (b, 0, n))],
        out_specs=pl.BlockSpec((1, _SOFT_DIM, tile), lambda b, n: (b, 0, n)),
        out_shape=jax.ShapeDtypeStruct((B, _SOFT_DIM, N), z.dtype),
    )(z)
